# Initial kernel scaffold; baseline (speedup 1.0000x reference)
import jax
import jax.numpy as jnp
from jax.experimental import pallas as pl


def kernel(x, edge_index, W1, b1, W2, b2, W3, b3):
    raise NotImplementedError("write your pallas kernel here")


# speedup vs baseline: 5.6591x; 5.6591x over previous
import functools

import jax
import jax.numpy as jnp
from jax import lax
from jax.experimental import pallas as pl
from jax.experimental.pallas import tpu as pltpu
from jax.experimental.pallas import tpu_sc as plsc

N = 10000
NPAD = 10112
E = 320000
K = 128
NCHUNK = E // K
NW = 32
NSUB = 16
TILE_ROWS = NPAD // NSUB
BM = 1264


def _row_chunks():
    out = []
    off = 0
    while off < TILE_ROWS:
        out.append((off, min(K, TILE_ROWS - off)))
        off += K
    return out


_CHUNKS = _row_chunks()


def _sc_cnt_body(dst_e, zerosw, onesw, cnt0, cnt1, didx, onesv, cntb, cntacc):
    c = lax.axis_index("c")
    s = lax.axis_index("s")
    wid = s * 2 + c
    base = s * TILE_ROWS

    pltpu.sync_copy(onesw, onesv)
    for off, sz in _CHUNKS:
        pltpu.sync_copy(zerosw.at[pl.ds(base + off, sz)], cntb.at[pl.ds(0, sz)])
        pltpu.sync_copy(cntb.at[pl.ds(0, sz)], cntacc.at[pl.ds(base + off, sz)])
    plsc.subcore_barrier()

    start = (wid * NCHUNK) // NW
    end = ((wid + 1) * NCHUNK) // NW

    def chunk(i, carry):
        off = pl.multiple_of(i * K, K)
        pltpu.sync_copy(dst_e.at[pl.ds(off, K)], didx)
        pltpu.sync_copy(onesv, cntacc.at[didx], add=True)
        return carry

    lax.fori_loop(start, end, chunk, 0)
    plsc.subcore_barrier()

    @pl.when(c == 0)
    def _wb0():
        for off, sz in _CHUNKS:
            pltpu.sync_copy(cntacc.at[pl.ds(base + off, sz)], cntb.at[pl.ds(0, sz)])
            pltpu.sync_copy(cntb.at[pl.ds(0, sz)], cnt0.at[pl.ds(base + off, sz)])

    @pl.when(c == 1)
    def _wb1():
        for off, sz in _CHUNKS:
            pltpu.sync_copy(cntacc.at[pl.ds(base + off, sz)], cntb.at[pl.ds(0, sz)])
            pltpu.sync_copy(cntb.at[pl.ds(0, sz)], cnt1.at[pl.ds(base + off, sz)])


def _sc_adj_body(z, src_e, dst_e, out0, out1, sidx, didx, rows, sem, acc):
    c = lax.axis_index("c")
    s = lax.axis_index("s")
    wid = s * 2 + c
    base = s * TILE_ROWS

    for off, sz in _CHUNKS:
        pltpu.sync_copy(z.at[pl.ds(base + off, sz)], rows.at[pl.ds(0, sz)])
        pltpu.sync_copy(rows.at[pl.ds(0, sz)], acc.at[pl.ds(base + off, sz)])
    plsc.subcore_barrier()

    start = (wid * NCHUNK) // NW
    end = ((wid + 1) * NCHUNK) // NW

    def chunk(i, carry):
        off = pl.multiple_of(i * K, K)
        pltpu.sync_copy(src_e.at[pl.ds(off, K)], sidx)
        pltpu.sync_copy(dst_e.at[pl.ds(off, K)], didx)
        pltpu.async_copy(z.at[sidx], rows, sem).wait()
        pltpu.sync_copy(rows, acc.at[didx], add=True)
        return carry

    lax.fori_loop(start, end, chunk, 0)
    plsc.subcore_barrier()

    @pl.when(c == 0)
    def _wb0():
        for off, sz in _CHUNKS:
            pltpu.sync_copy(acc.at[pl.ds(base + off, sz)], rows.at[pl.ds(0, sz)])
            pltpu.sync_copy(rows.at[pl.ds(0, sz)], out0.at[pl.ds(base + off, sz)])

    @pl.when(c == 1)
    def _wb1():
        for off, sz in _CHUNKS:
            pltpu.sync_copy(acc.at[pl.ds(base + off, sz)], rows.at[pl.ds(0, sz)])
            pltpu.sync_copy(rows.at[pl.ds(0, sz)], out1.at[pl.ds(base + off, sz)])


def _sc_mesh():
    return plsc.VectorSubcoreMesh(
        core_axis_name="c", subcore_axis_name="s", num_cores=2, num_subcores=16
    )


@functools.lru_cache(maxsize=None)
def _make_sc_cnt():
    return pl.kernel(
        _sc_cnt_body,
        out_type=[jax.ShapeDtypeStruct((NPAD, 128), jnp.float32)] * 2,
        mesh=_sc_mesh(),
        scratch_types=[
            pltpu.VMEM((K,), jnp.int32),
            pltpu.VMEM((K, 128), jnp.float32),
            pltpu.VMEM((K, 128), jnp.float32),
            pltpu.VMEM_SHARED((NPAD, 128), jnp.float32),
        ],
    )


@functools.lru_cache(maxsize=None)
def _make_sc_adj():
    return pl.kernel(
        _sc_adj_body,
        out_type=[jax.ShapeDtypeStruct((NPAD, 128), jnp.float32)] * 2,
        mesh=_sc_mesh(),
        scratch_types=[
            pltpu.VMEM((K,), jnp.int32),
            pltpu.VMEM((K,), jnp.int32),
            pltpu.VMEM((K, 128), jnp.float32),
            pltpu.SemaphoreType.DMA,
            pltpu.VMEM_SHARED((NPAD, 128), jnp.float32),
        ],
    )


def _lin_body(x_ref, w_ref, b_ref, o_ref):
    o_ref[...] = (
        jnp.dot(x_ref[...], w_ref[...], preferred_element_type=jnp.float32)
        + b_ref[...]
    )


def _tc_linear(x, W, b, bm=BM):
    m, d = x.shape
    h = W.shape[1]
    return pl.pallas_call(
        _lin_body,
        grid=(m // bm,),
        in_specs=[
            pl.BlockSpec((bm, d), lambda i: (i, 0)),
            pl.BlockSpec((d, h), lambda i: (0, 0)),
            pl.BlockSpec((1, h), lambda i: (0, 0)),
        ],
        out_specs=pl.BlockSpec((bm, h), lambda i: (i, 0)),
        out_shape=jax.ShapeDtypeStruct((m, h), jnp.float32),
    )(x, W, b.reshape(1, -1))


def _merge_rows(p0_ref, p1_ref, z_ref, c0_ref, c1_ref):
    u = jnp.maximum(p0_ref[...] + p1_ref[...] - z_ref[...], 0.0)
    inv = 1.0 / (1.0 + c0_ref[...][:, 0:1] + c1_ref[...][:, 0:1])
    return u * inv


def _mid_body(p0_ref, p1_ref, z_ref, c0_ref, c1_ref, w_ref, b_ref, o_ref):
    hrows = _merge_rows(p0_ref, p1_ref, z_ref, c0_ref, c1_ref)
    o_ref[...] = (
        jnp.dot(hrows, w_ref[...], preferred_element_type=jnp.float32)
        + b_ref[...]
    )


def _out_body(p0_ref, p1_ref, z_ref, c0_ref, c1_ref, w_ref, b_ref, o_ref):
    hrows = _merge_rows(p0_ref, p1_ref, z_ref, c0_ref, c1_ref)
    t = (
        jnp.dot(hrows, w_ref[...], preferred_element_type=jnp.float32)
        + b_ref[...]
    )
    mx = jnp.max(t, axis=1, keepdims=True)
    lse = jnp.log(jnp.sum(jnp.exp(t - mx), axis=1, keepdims=True)) + mx
    o_ref[...] = t - lse


def _tc_merge_mm(body, p0, p1, z, c0, c1, W, b, bm=BM):
    m, d = z.shape
    h = W.shape[1]
    return pl.pallas_call(
        body,
        grid=(m // bm,),
        in_specs=[
            pl.BlockSpec((bm, d), lambda i: (i, 0)),
            pl.BlockSpec((bm, d), lambda i: (i, 0)),
            pl.BlockSpec((bm, d), lambda i: (i, 0)),
            pl.BlockSpec((bm, d), lambda i: (i, 0)),
            pl.BlockSpec((bm, d), lambda i: (i, 0)),
            pl.BlockSpec((d, h), lambda i: (0, 0)),
            pl.BlockSpec((1, h), lambda i: (0, 0)),
        ],
        out_specs=pl.BlockSpec((bm, h), lambda i: (i, 0)),
        out_shape=jax.ShapeDtypeStruct((m, h), jnp.float32),
    )(p0, p1, z, c0, c1, W, b.reshape(1, -1))


def kernel(x, edge_index, W1, b1, W2, b2, W3, b3):
    sc_cnt = _make_sc_cnt()
    sc_adj = _make_sc_adj()

    dst_e = edge_index[0]
    src_e = edge_index[1]
    zerosw = jnp.zeros((NPAD, 128), jnp.float32)
    onesw = jnp.ones((K, 128), jnp.float32)

    xp = jnp.pad(x, ((0, NPAD - N), (0, 0)))
    c0, c1 = sc_cnt(dst_e, zerosw, onesw)
    z1 = _tc_linear(xp, W1, b1)
    p0, p1 = sc_adj(z1, src_e, dst_e)
    z2 = _tc_merge_mm(_mid_body, p0, p1, z1, c0, c1, W2, b2)
    q0, q1 = sc_adj(z2, src_e, dst_e)
    return _tc_merge_mm(_out_body, q0, q1, z2, c0, c1, W3, b3)[:N]
